# baseline (speedup 1.0000x reference)
import functools

import jax
import jax.numpy as jnp
from jax import lax
from jax.experimental import pallas as pl
from jax.experimental.pallas import tpu as pltpu
from jax.experimental.pallas import tpu_sc as plsc

NUM_CLASS = 512
NUM_EMB = 8192
EMB_DIM = 64
TOP_K = 3
COMMITMENT = 0.25
T_PLUS_1 = 129
B = 32

_IDX_PAD = 128


def _vq_body(x_ref, w_ref, w2_ref, idx_ref, dsum_ref, counts_ref,
             *, n_codes, offset):
    i = pl.program_id(0)
    x = x_ref[...]
    w = w_ref[...]
    w2 = w2_ref[...]
    r = x.shape[0]

    x2 = jnp.sum(x * x, axis=1, keepdims=True)
    xw2 = lax.dot_general(x * -2.0, w, (((1,), (1,)), ((), ())),
                          preferred_element_type=jnp.float32,
                          precision=lax.Precision.DEFAULT)
    d = (x2 + w2) + xw2

    colf = lax.broadcasted_iota(jnp.int32, (r, n_codes), 1).astype(jnp.float32)
    lane = lax.broadcasted_iota(jnp.int32, (r, _IDX_PAD), 1)
    inf = jnp.float32(jnp.inf)
    bigf = jnp.float32(n_codes)

    idx_out = jnp.zeros((r, _IDX_PAD), jnp.int32)
    dsum = jnp.zeros((r, 1), jnp.float32)
    for k in range(TOP_K):
        mv = jnp.min(d, axis=1, keepdims=True)
        ikf = jnp.min(jnp.where(d == mv, colf, bigf), axis=1,
                      keepdims=True)
        d = jnp.where(colf == ikf, inf, d)
        ik = ikf.astype(jnp.int32)
        idx_out = jnp.where(lane == k, ik + offset, idx_out)
        dsum = dsum + mv

    idx_ref[...] = idx_out
    counts_part = jnp.sum(jnp.where(d == inf, 1.0, 0.0), axis=0,
                          keepdims=True)

    @pl.when(i == 0)
    def _init():
        dsum_ref[...] = jnp.zeros_like(dsum_ref)
        counts_ref[...] = jnp.zeros_like(counts_ref)

    dsum_ref[...] += jnp.sum(dsum)[None, None]
    counts_ref[...] += counts_part


def _vq_topk(x, w, w2, *, block_rows, offset, interpret=False):
    n, c = x.shape[0], w.shape[0]
    grid = n // block_rows
    body = functools.partial(_vq_body, n_codes=c, offset=offset)
    return pl.pallas_call(
        body,
        grid=(grid,),
        in_specs=[
            pl.BlockSpec((block_rows, EMB_DIM), lambda i: (i, 0)),
            pl.BlockSpec((c, EMB_DIM), lambda i: (0, 0)),
            pl.BlockSpec((1, c), lambda i: (0, 0)),
        ],
        out_specs=[
            pl.BlockSpec((block_rows, _IDX_PAD), lambda i: (i, 0)),
            pl.BlockSpec((1, 1), lambda i: (0, 0)),
            pl.BlockSpec((1, c), lambda i: (0, 0)),
        ],
        out_shape=[
            jax.ShapeDtypeStruct((n, _IDX_PAD), jnp.int32),
            jax.ShapeDtypeStruct((1, 1), jnp.float32),
            jax.ShapeDtypeStruct((1, c), jnp.float32),
        ],
        interpret=interpret,
    )(x, w, w2)


def _finish_body(counts_f_ref, counts_c_ref, dsum_f_ref, dsum_c_ref,
                 loss_ref, fperp_ref, cperp_ref):
    n_f = jnp.float32((T_PLUS_1 - 1) * B)
    n_c = jnp.float32(B)
    mean_f = dsum_f_ref[0, 0] / (n_f * TOP_K * EMB_DIM)
    mean_c = dsum_c_ref[0, 0] / (n_c * TOP_K * EMB_DIM)
    loss = (mean_f + COMMITMENT * mean_f) + (mean_c + COMMITMENT * mean_c)
    loss_ref[...] = loss[None, None]

    p_f = counts_f_ref[...] / n_f
    p_c = counts_c_ref[...] / n_c
    ent_f = jnp.sum(p_f * jnp.log(p_f + 1e-10))
    ent_c = jnp.sum(p_c * jnp.log(p_c + 1e-10))
    fperp_ref[...] = jnp.exp(-ent_f)[None, None]
    cperp_ref[...] = jnp.exp(-ent_c)[None, None]


def _finish(counts_f, counts_c, dsum_f, dsum_c, interpret=False):
    return pl.pallas_call(
        _finish_body,
        out_shape=[
            jax.ShapeDtypeStruct((1, 1), jnp.float32),
            jax.ShapeDtypeStruct((1, 1), jnp.float32),
            jax.ShapeDtypeStruct((1, 1), jnp.float32),
        ],
        interpret=interpret,
    )(counts_f, counts_c, dsum_f, dsum_c)


_GATHER_PAD = 16384
_CHUNK = 128
_ROW_PAD = 128
_PAIRS = (NUM_CLASS + NUM_EMB) // 2
_HALF = _PAIRS // 2


def _sc_gather(table2, idx_lo2, idx_hi2):
    info = plsc.get_sparse_core_info()
    ns = info.num_subcores
    rows_per_w = _GATHER_PAD // ns
    n_chunks = rows_per_w // _CHUNK
    wave = n_chunks // 2
    stripe = _HALF // ns
    mesh = plsc.VectorSubcoreMesh(core_axis_name="c", subcore_axis_name="s")

    @functools.partial(
        pl.kernel, mesh=mesh,
        out_type=[jax.ShapeDtypeStruct((_GATHER_PAD, _ROW_PAD), jnp.float32),
                  jax.ShapeDtypeStruct((_GATHER_PAD, _ROW_PAD), jnp.float32)],
        scratch_types=[
            pltpu.VMEM((n_chunks, _CHUNK), jnp.int32),
            pltpu.VMEM((wave * _CHUNK, _ROW_PAD), jnp.float32),
            pltpu.VMEM((stripe, _ROW_PAD), jnp.float32),
            pltpu.VMEM_SHARED((_HALF, _ROW_PAD), jnp.float32),
            pltpu.SemaphoreType.DMA,
        ],
    )
    def gk(t_hbm, ilo_hbm, ihi_hbm, out0_hbm, out1_hbm,
           idx_v, rows_v, stage_v, tsh, sem):
        cid = lax.axis_index("c")
        sid = lax.axis_index("s")
        pltpu.sync_copy(t_hbm.at[pl.ds(cid * _HALF + sid * stripe, stripe)],
                        stage_v)
        pltpu.sync_copy(stage_v, tsh.at[pl.ds(sid * stripe, stripe)])

        @pl.when(cid == 0)
        def _i0():
            pltpu.sync_copy(ilo_hbm.at[pl.ds(sid * n_chunks, n_chunks)], idx_v)

        @pl.when(cid == 1)
        def _i1():
            pltpu.sync_copy(ihi_hbm.at[pl.ds(sid * n_chunks, n_chunks)], idx_v)

        plsc.subcore_barrier()

        for w in range(2):
            cps = [pltpu.async_copy(tsh.at[idx_v.at[w * wave + c]],
                                    rows_v.at[pl.ds(c * _CHUNK, _CHUNK)], sem)
                   for c in range(wave)]
            for cp in cps:
                cp.wait()
            off = sid * rows_per_w + w * wave * _CHUNK

            @pl.when(cid == 0)
            def _w0():
                pltpu.sync_copy(rows_v, out0_hbm.at[pl.ds(off, wave * _CHUNK)])

            @pl.when(cid == 1)
            def _w1():
                pltpu.sync_copy(rows_v, out1_hbm.at[pl.ds(off, wave * _CHUNK)])

    return gk(table2, idx_lo2, idx_hi2)


def kernel(features, class_codebook, feature_codebook):
    xc = features[0]
    xf = features[1:].reshape(-1, EMB_DIM)
    w2f = jnp.sum(feature_codebook * feature_codebook, axis=1)[None]
    w2c = jnp.sum(class_codebook * class_codebook, axis=1)[None]

    idx_f_pad, dsum_f, counts_f = _vq_topk(
        xf, feature_codebook, w2f, block_rows=256, offset=NUM_CLASS)
    idx_c_pad, dsum_c, counts_c = _vq_topk(
        xc, class_codebook, w2c, block_rows=32, offset=0)

    loss, fperp, cperp = _finish(counts_f, counts_c, dsum_f, dsum_c)

    enc = jnp.concatenate(
        [idx_c_pad[:, :TOP_K].reshape(1, B, TOP_K),
         idx_f_pad[:, :TOP_K].reshape(T_PLUS_1 - 1, B, TOP_K)], axis=0)

    table2 = jnp.concatenate([class_codebook, feature_codebook],
                             axis=0).reshape(_PAIRS, _ROW_PAD)
    flat_idx = enc.reshape(-1)
    flat_idx = jnp.concatenate(
        [flat_idx,
         jnp.zeros((_GATHER_PAD - flat_idx.shape[0],), jnp.int32)])
    pair_idx = flat_idx >> 1
    idx_lo = jnp.minimum(pair_idx, _HALF - 1)
    idx_hi = jnp.clip(pair_idx - _HALF, 0, _HALF - 1)
    out0, out1 = _sc_gather(table2,
                            idx_lo.reshape(_GATHER_PAD // _CHUNK, _CHUNK),
                            idx_hi.reshape(_GATHER_PAD // _CHUNK, _CHUNK))
    n_real = T_PLUS_1 * B * TOP_K
    shape4 = (T_PLUS_1, B, TOP_K, EMB_DIM)
    sel_hi = (pair_idx[:n_real] >= _HALF).reshape(T_PLUS_1, B, TOP_K, 1)
    odd = ((flat_idx[:n_real] & 1) == 1).reshape(T_PLUS_1, B, TOP_K, 1)
    p0 = out0[:n_real].reshape(T_PLUS_1, B, TOP_K, 2, EMB_DIM)
    p1 = out1[:n_real].reshape(T_PLUS_1, B, TOP_K, 2, EMB_DIM)
    q = jnp.where(
        sel_hi,
        jnp.where(odd, p1[..., 1, :], p1[..., 0, :]),
        jnp.where(odd, p0[..., 1, :], p0[..., 0, :])).reshape(shape4)

    return (loss.reshape(()), q, fperp.reshape(()), cperp.reshape(()), enc)


# speedup vs baseline: 1.7219x; 1.7219x over previous
import functools

import jax
import jax.numpy as jnp
from jax import lax
from jax.experimental import pallas as pl
from jax.experimental.pallas import tpu as pltpu
from jax.experimental.pallas import tpu_sc as plsc

NUM_CLASS = 512
NUM_EMB = 8192
EMB_DIM = 64
TOP_K = 3
COMMITMENT = 0.25
T_PLUS_1 = 129
B = 32

_IDX_PAD = 128


def _vq_body(x_ref, w_ref, w2_ref, idx_ref, dsum_ref, counts_ref,
             *, n_codes, offset):
    i = pl.program_id(0)
    x = x_ref[...]
    w = w_ref[...]
    w2 = w2_ref[...]
    r = x.shape[0]

    x2 = jnp.sum(x * x, axis=1, keepdims=True)
    xw2 = lax.dot_general(x * -2.0, w, (((1,), (1,)), ((), ())),
                          preferred_element_type=jnp.float32,
                          precision=lax.Precision.DEFAULT)
    d = (x2 + w2) + xw2

    colf = lax.broadcasted_iota(jnp.int32, (r, n_codes), 1).astype(jnp.float32)
    lane = lax.broadcasted_iota(jnp.int32, (r, _IDX_PAD), 1)
    inf = jnp.float32(jnp.inf)
    bigf = jnp.float32(n_codes)

    idx_out = jnp.zeros((r, _IDX_PAD), jnp.int32)
    dsum = jnp.zeros((r, 1), jnp.float32)
    for k in range(TOP_K):
        mv = jnp.min(d, axis=1, keepdims=True)
        ikf = jnp.min(jnp.where(d == mv, colf, bigf), axis=1,
                      keepdims=True)
        d = jnp.where(colf == ikf, inf, d)
        ik = ikf.astype(jnp.int32)
        idx_out = jnp.where(lane == k, ik + offset, idx_out)
        dsum = dsum + mv

    idx_ref[...] = idx_out
    counts_part = jnp.sum(jnp.where(d == inf, 1.0, 0.0), axis=0,
                          keepdims=True)

    @pl.when(i == 0)
    def _init():
        dsum_ref[...] = jnp.zeros_like(dsum_ref)
        counts_ref[...] = jnp.zeros_like(counts_ref)

    dsum_ref[...] += jnp.sum(dsum)[None, None]
    counts_ref[...] += counts_part


def _vq_topk(x, w, w2, *, block_rows, offset, interpret=False):
    n, c = x.shape[0], w.shape[0]
    grid = n // block_rows
    body = functools.partial(_vq_body, n_codes=c, offset=offset)
    return pl.pallas_call(
        body,
        grid=(grid,),
        in_specs=[
            pl.BlockSpec((block_rows, EMB_DIM), lambda i: (i, 0)),
            pl.BlockSpec((c, EMB_DIM), lambda i: (0, 0)),
            pl.BlockSpec((1, c), lambda i: (0, 0)),
        ],
        out_specs=[
            pl.BlockSpec((block_rows, _IDX_PAD), lambda i: (i, 0)),
            pl.BlockSpec((1, 1), lambda i: (0, 0)),
            pl.BlockSpec((1, c), lambda i: (0, 0)),
        ],
        out_shape=[
            jax.ShapeDtypeStruct((n, _IDX_PAD), jnp.int32),
            jax.ShapeDtypeStruct((1, 1), jnp.float32),
            jax.ShapeDtypeStruct((1, c), jnp.float32),
        ],
        interpret=interpret,
    )(x, w, w2)


def _finish_body(counts_f_ref, counts_c_ref, dsum_f_ref, dsum_c_ref,
                 loss_ref, fperp_ref, cperp_ref):
    n_f = jnp.float32((T_PLUS_1 - 1) * B)
    n_c = jnp.float32(B)
    mean_f = dsum_f_ref[0, 0] / (n_f * TOP_K * EMB_DIM)
    mean_c = dsum_c_ref[0, 0] / (n_c * TOP_K * EMB_DIM)
    loss = (mean_f + COMMITMENT * mean_f) + (mean_c + COMMITMENT * mean_c)
    loss_ref[...] = loss[None, None]

    p_f = counts_f_ref[...] / n_f
    p_c = counts_c_ref[...] / n_c
    ent_f = jnp.sum(p_f * jnp.log(p_f + 1e-10))
    ent_c = jnp.sum(p_c * jnp.log(p_c + 1e-10))
    fperp_ref[...] = jnp.exp(-ent_f)[None, None]
    cperp_ref[...] = jnp.exp(-ent_c)[None, None]


def _finish(counts_f, counts_c, dsum_f, dsum_c, interpret=False):
    return pl.pallas_call(
        _finish_body,
        out_shape=[
            jax.ShapeDtypeStruct((1, 1), jnp.float32),
            jax.ShapeDtypeStruct((1, 1), jnp.float32),
            jax.ShapeDtypeStruct((1, 1), jnp.float32),
        ],
        interpret=interpret,
    )(counts_f, counts_c, dsum_f, dsum_c)


_GATHER_PAD = 16384
_CHUNK = 128
_ROW_PAD = 128
_PAIRS = (NUM_CLASS + NUM_EMB) // 2
_HALF = _PAIRS // 2


def _sc_gather(table2, idx_lo2, idx_hi2):
    info = plsc.get_sparse_core_info()
    ns = info.num_subcores
    rows_per_w = _GATHER_PAD // ns
    n_chunks = rows_per_w // _CHUNK
    wave = n_chunks // 2
    stripe = _HALF // ns
    mesh = plsc.VectorSubcoreMesh(core_axis_name="c", subcore_axis_name="s")

    @functools.partial(
        pl.kernel, mesh=mesh,
        out_type=[jax.ShapeDtypeStruct((_GATHER_PAD, _ROW_PAD), jnp.float32),
                  jax.ShapeDtypeStruct((_GATHER_PAD, _ROW_PAD), jnp.float32)],
        scratch_types=[
            pltpu.VMEM((n_chunks, _CHUNK), jnp.int32),
            pltpu.VMEM((wave * _CHUNK, _ROW_PAD), jnp.float32),
            pltpu.VMEM((stripe, _ROW_PAD), jnp.float32),
            pltpu.VMEM_SHARED((_HALF, _ROW_PAD), jnp.float32),
            pltpu.SemaphoreType.DMA,
        ],
    )
    def gk(t_hbm, ilo_hbm, ihi_hbm, out0_hbm, out1_hbm,
           idx_v, rows_v, stage_v, tsh, sem):
        cid = lax.axis_index("c")
        sid = lax.axis_index("s")
        pltpu.sync_copy(t_hbm.at[pl.ds(cid * _HALF + sid * stripe, stripe)],
                        stage_v)
        pltpu.sync_copy(stage_v, tsh.at[pl.ds(sid * stripe, stripe)])

        @pl.when(cid == 0)
        def _i0():
            pltpu.sync_copy(ilo_hbm.at[pl.ds(sid * n_chunks, n_chunks)], idx_v)

        @pl.when(cid == 1)
        def _i1():
            pltpu.sync_copy(ihi_hbm.at[pl.ds(sid * n_chunks, n_chunks)], idx_v)

        plsc.subcore_barrier()

        for w in range(2):
            cps = [pltpu.async_copy(tsh.at[idx_v.at[w * wave + c]],
                                    rows_v.at[pl.ds(c * _CHUNK, _CHUNK)], sem)
                   for c in range(wave)]
            for cp in cps:
                cp.wait()
            off = sid * rows_per_w + w * wave * _CHUNK

            @pl.when(cid == 0)
            def _w0():
                pltpu.sync_copy(rows_v, out0_hbm.at[pl.ds(off, wave * _CHUNK)])

            @pl.when(cid == 1)
            def _w1():
                pltpu.sync_copy(rows_v, out1_hbm.at[pl.ds(off, wave * _CHUNK)])

    return gk(table2, idx_lo2, idx_hi2)


def kernel(features, class_codebook, feature_codebook):
    xc = features[0]
    xf = features[1:].reshape(-1, EMB_DIM)
    w2f = jnp.sum(feature_codebook * feature_codebook, axis=1)[None]
    w2c = jnp.sum(class_codebook * class_codebook, axis=1)[None]

    idx_f_pad, dsum_f, counts_f = _vq_topk(
        xf, feature_codebook, w2f, block_rows=512, offset=NUM_CLASS)
    idx_c_pad, dsum_c, counts_c = _vq_topk(
        xc, class_codebook, w2c, block_rows=32, offset=0)

    loss, fperp, cperp = _finish(counts_f, counts_c, dsum_f, dsum_c)

    enc = jnp.concatenate(
        [idx_c_pad[:, :TOP_K].reshape(1, B, TOP_K),
         idx_f_pad[:, :TOP_K].reshape(T_PLUS_1 - 1, B, TOP_K)], axis=0)

    table2 = jnp.concatenate([class_codebook, feature_codebook],
                             axis=0).reshape(_PAIRS, _ROW_PAD)
    flat_idx = enc.reshape(-1)
    flat_idx = jnp.concatenate(
        [flat_idx,
         jnp.zeros((_GATHER_PAD - flat_idx.shape[0],), jnp.int32)])
    pair_idx = flat_idx >> 1
    idx_lo = jnp.minimum(pair_idx, _HALF - 1)
    idx_hi = jnp.clip(pair_idx - _HALF, 0, _HALF - 1)
    out0, out1 = _sc_gather(table2,
                            idx_lo.reshape(_GATHER_PAD // _CHUNK, _CHUNK),
                            idx_hi.reshape(_GATHER_PAD // _CHUNK, _CHUNK))
    n_real = T_PLUS_1 * B * TOP_K
    sel_hi = (pair_idx[:n_real] >= _HALF)[:, None]
    odd = (flat_idx[:n_real] & 1)[:, None] == 1
    q = jnp.where(
        sel_hi,
        jnp.where(odd, out1[:n_real, EMB_DIM:], out1[:n_real, :EMB_DIM]),
        jnp.where(odd, out0[:n_real, EMB_DIM:], out0[:n_real, :EMB_DIM]))
    q = q.reshape(T_PLUS_1, B, TOP_K, EMB_DIM)

    return (loss.reshape(()), q, fperp.reshape(()), cperp.reshape(()), enc)
